# baseline (speedup 1.0000x reference)
import functools

import jax
import jax.numpy as jnp
from jax.experimental import pallas as pl
from jax.experimental.pallas import tpu as pltpu


def _masked_mm_kernel(mask_ref, fact0_ref, xw_ref, inp_ref, params_ref,
                      bias_ref, out_ref, *, num_ids, order_static):
    tf = out_ref.shape[0]
    fp = jnp.ones_like(inp_ref[0])
    for i in range(order_static):
        m = mask_ref[i]
        fp = fp * (inp_ref[i] * m + (1.0 - m))
    f0 = fact0_ref[:]
    npad = xw_ref.shape[0]
    iota_n = jax.lax.broadcasted_iota(jnp.int32, (tf, npad), 1)
    onehot_n = (iota_n == f0[:, None]).astype(jnp.float32)
    ids_f = jnp.sum(onehot_n * xw_ref[:][None, :], axis=1)
    ids_i = ids_f.astype(jnp.int32)
    ppad = bias_ref.shape[0]
    iota_p = jax.lax.broadcasted_iota(jnp.int32, (tf, ppad), 1)
    onehot_p = (iota_p == ids_i[:, None]).astype(jnp.float32)
    acc = jnp.dot(onehot_p, bias_ref[:], preferred_element_type=jnp.float32)

    def body(p, acc):
        m = (ids_i == p).astype(jnp.float32)
        w = params_ref[p]
        fpm = (fp * m[:, None]).astype(jnp.bfloat16)
        return acc + jnp.dot(fpm, w, preferred_element_type=jnp.float32)

    acc = jax.lax.fori_loop(0, num_ids, body, acc)
    out_ref[...] = acc


def kernel(x, fact, inp, msg_to, order, params, bias):
    num_ids, H, O = params.shape
    order_static, F, _ = inp.shape
    n_rows = x.shape[0]
    m_atoms = int(round(float(num_ids) ** 0.5))

    idx = jnp.arange(order_static)
    mask = ((idx < order) & (idx != msg_to)).astype(jnp.float32)

    xw = (x[:, 1] * m_atoms + x[:, 2]).astype(jnp.float32)
    npad = ((n_rows + 127) // 128) * 128
    xw_pad = jnp.zeros((npad,), jnp.float32).at[:n_rows].set(xw)
    fact0 = fact[:, 0].astype(jnp.int32)

    ppad = ((num_ids + 127) // 128) * 128
    bias_pad = jnp.zeros((ppad, O), jnp.float32).at[:num_ids].set(
        bias.reshape(num_ids, O))

    TF = 512
    grid = (F // TF,)
    out = pl.pallas_call(
        functools.partial(_masked_mm_kernel, num_ids=num_ids,
                          order_static=order_static),
        grid=grid,
        in_specs=[
            pl.BlockSpec(memory_space=pltpu.SMEM),
            pl.BlockSpec((TF,), lambda t: (t,)),
            pl.BlockSpec((npad,), lambda t: (0,)),
            pl.BlockSpec((order_static, TF, H), lambda t: (0, t, 0)),
            pl.BlockSpec((num_ids, H, O), lambda t: (0, 0, 0)),
            pl.BlockSpec((ppad, O), lambda t: (0, 0)),
        ],
        out_specs=pl.BlockSpec((TF, O), lambda t: (t, 0)),
        out_shape=jax.ShapeDtypeStruct((F, O), jnp.float32),
    )(mask, fact0, xw_pad, inp, params.astype(jnp.bfloat16), bias_pad)
    return out


# speedup vs baseline: 4.6167x; 4.6167x over previous
import functools

import jax
import jax.numpy as jnp
from jax import lax
from jax.experimental import pallas as pl
from jax.experimental.pallas import tpu as pltpu
from jax.experimental.pallas import tpu_sc as plsc

_NC = 2
_NS = 16
_NW = _NC * _NS


def _prep_kernel(mask_ref, fact0_ref, xw_ref, inp_ref,
                 fp_ref, ids_ref, offs_ref, tlohi_ref, hist_ref,
                 *, order_static, num_tiles, tile_rows):
    t = pl.program_id(0)

    @pl.when(t == 0)
    def _():
        hist_ref[...] = jnp.zeros_like(hist_ref)

    tf = fp_ref.shape[0]
    fp = jnp.ones_like(inp_ref[0])
    for i in range(order_static):
        m = mask_ref[i]
        fp = fp * (inp_ref[i] * m + (1.0 - m))
    fp_ref[...] = fp

    f0 = fact0_ref[:]
    npad = xw_ref.shape[0]
    iota_n = lax.broadcasted_iota(jnp.int32, (tf, npad), 1)
    onehot_n = (iota_n == f0[:, None]).astype(jnp.float32)
    ids_f = jnp.sum(onehot_n * xw_ref[:][None, :], axis=1)
    ids_i = ids_f.astype(jnp.int32)
    ids_ref[...] = ids_i

    ppad = offs_ref.shape[0]
    iota_p = lax.broadcasted_iota(jnp.int32, (tf, ppad), 1)
    oh = (iota_p == ids_i[:, None]).astype(jnp.float32)
    hist_ref[0, :] = hist_ref[0, :] + jnp.sum(oh, axis=0)

    @pl.when(t == num_tiles - 1)
    def _():
        hist = hist_ref[0, :]
        r = lax.broadcasted_iota(jnp.int32, (ppad, ppad), 0)
        c = lax.broadcasted_iota(jnp.int32, (ppad, ppad), 1)
        ut = (r < c).astype(jnp.float32)
        offs_f = jnp.dot(hist[None, :], ut,
                         preferred_element_type=jnp.float32)[0]
        offs_ref[...] = offs_f.astype(jnp.int32)
        tvals = lax.broadcasted_iota(jnp.int32, (128, ppad), 0) * tile_rows
        lo = jnp.sum((offs_f[None, :] <= tvals.astype(jnp.float32)),
                     axis=1).astype(jnp.int32) - 1
        hiv = jnp.sum((offs_f[None, :] <=
                       (tvals + (tile_rows - 1)).astype(jnp.float32)),
                      axis=1).astype(jnp.int32) - 1
        tlohi_ref[...] = jnp.concatenate([lo[None, :], hiv[None, :]], axis=0)


def _pos_kernel(ids_ref, offs_ref, pos_ref, run_ref, *, num_tiles):
    t = pl.program_id(0)

    @pl.when(t == 0)
    def _():
        run_ref[...] = jnp.zeros_like(run_ref)

    tf = ids_ref.shape[0]
    ppad = offs_ref.shape[0]
    ids_i = ids_ref[:]
    iota_p = lax.broadcasted_iota(jnp.int32, (tf, ppad), 1)
    oh = (iota_p == ids_i[:, None]).astype(jnp.float32)

    offs_f = offs_ref[:].astype(jnp.float32)
    sub = 128
    nsub = tf // sub
    r = lax.broadcasted_iota(jnp.int32, (sub, sub), 0)
    c = lax.broadcasted_iota(jnp.int32, (sub, sub), 1)
    lt = (c < r).astype(jnp.bfloat16)
    run = run_ref[0, :]
    for s in range(nsub):
        ohs = oh[s * sub:(s + 1) * sub]
        ms = jnp.dot(lt, ohs.astype(jnp.bfloat16),
                     preferred_element_type=jnp.float32)
        posv = jnp.sum((ms + (run + offs_f)[None, :]) * ohs, axis=1)
        pos_ref[0, s, :] = posv.astype(jnp.int32)
        run = run + jnp.sum(ohs, axis=0)
    run_ref[0, :] = run


def _make_permute_rows(F, D, gather):
    chunk = F // _NW
    k = chunk // 128
    mesh = plsc.VectorSubcoreMesh(core_axis_name="c", subcore_axis_name="s")

    @functools.partial(
        pl.kernel, mesh=mesh,
        out_type=jax.ShapeDtypeStruct((F, D), jnp.float32),
        scratch_types=[
            pltpu.VMEM((k, 128), jnp.int32),
            pltpu.VMEM((chunk, D), jnp.float32),
            pltpu.SemaphoreType.DMA,
        ],
    )
    def permute(rows_hbm, pos_hbm, out_hbm, idx_v, rows_v, sem):
        wid = lax.axis_index("s") * _NC + lax.axis_index("c")
        base = wid * chunk
        pltpu.sync_copy(pos_hbm.at[pl.ds(wid * k, k)], idx_v)
        if gather:
            for j in range(k):
                pltpu.async_copy(rows_hbm.at[idx_v.at[j]],
                                 rows_v.at[pl.ds(j * 128, 128)], sem).wait()
            pltpu.sync_copy(rows_v, out_hbm.at[pl.ds(base, chunk)])
        else:
            pltpu.sync_copy(rows_hbm.at[pl.ds(base, chunk)], rows_v)
            for j in range(k):
                pltpu.async_copy(rows_v.at[pl.ds(j * 128, 128)],
                                 out_hbm.at[idx_v.at[j]], sem).wait()

    return permute


def _seg_mm_kernel(offs_ref, tlohi_ref, fps_ref, params_ref, bias_ref,
                   out_ref, *, tile_rows):
    t = pl.program_id(0)
    lo = tlohi_ref[0, t]
    hi = tlohi_ref[1, t]
    fpsb = fps_ref[...].astype(jnp.bfloat16)
    r_glob = lax.broadcasted_iota(jnp.int32, (tile_rows,), 0) + t * tile_rows
    acc0 = jnp.zeros((tile_rows, bias_ref.shape[1]), jnp.float32)

    def body(p, acc):
        o0 = offs_ref[p]
        o1 = offs_ref[p + 1]
        m = (r_glob >= o0) & (r_glob < o1)
        mf = m.astype(jnp.float32)
        fpm = fpsb * m.astype(jnp.bfloat16)[:, None]
        acc = acc + jnp.dot(fpm, params_ref[p],
                            preferred_element_type=jnp.float32)
        return acc + mf[:, None] * bias_ref[p][None, :]

    out_ref[...] = lax.fori_loop(lo, hi + 1, body, acc0)


def kernel(x, fact, inp, msg_to, order, params, bias):
    num_ids, H, O = params.shape
    order_static, F, _ = inp.shape
    n_rows = x.shape[0]
    m_atoms = int(round(float(num_ids) ** 0.5))

    idx = jnp.arange(order_static)
    mask = ((idx < order) & (idx != msg_to)).astype(jnp.float32)

    xw = (x[:, 1] * m_atoms + x[:, 2]).astype(jnp.float32)
    npad = ((n_rows + 127) // 128) * 128
    xw_pad = jnp.zeros((npad,), jnp.float32).at[:n_rows].set(xw)
    fact0 = fact[:, 0].astype(jnp.int32)

    ppad = 256
    TF = 512
    nt = F // TF

    fp, ids, offs, tlohi = pl.pallas_call(
        functools.partial(_prep_kernel, order_static=order_static,
                          num_tiles=nt, tile_rows=TF),
        grid=(nt,),
        in_specs=[
            pl.BlockSpec(memory_space=pltpu.SMEM),
            pl.BlockSpec((TF,), lambda t: (t,)),
            pl.BlockSpec((npad,), lambda t: (0,)),
            pl.BlockSpec((order_static, TF, H), lambda t: (0, t, 0)),
        ],
        out_specs=[
            pl.BlockSpec((TF, H), lambda t: (t, 0)),
            pl.BlockSpec((TF,), lambda t: (t,)),
            pl.BlockSpec((ppad,), lambda t: (0,)),
            pl.BlockSpec((2, 128), lambda t: (0, 0)),
        ],
        out_shape=[
            jax.ShapeDtypeStruct((F, H), jnp.float32),
            jax.ShapeDtypeStruct((F,), jnp.int32),
            jax.ShapeDtypeStruct((ppad,), jnp.int32),
            jax.ShapeDtypeStruct((2, 128), jnp.int32),
        ],
        scratch_shapes=[pltpu.VMEM((8, ppad), jnp.float32)],
    )(mask, fact0, xw_pad, inp)

    pos2d = pl.pallas_call(
        functools.partial(_pos_kernel, num_tiles=nt),
        grid=(nt,),
        in_specs=[
            pl.BlockSpec((TF,), lambda t: (t,)),
            pl.BlockSpec((ppad,), lambda t: (0,)),
        ],
        out_specs=pl.BlockSpec((1, TF // 128, 128), lambda t: (t, 0, 0)),
        out_shape=jax.ShapeDtypeStruct((nt, TF // 128, 128), jnp.int32),
        scratch_shapes=[pltpu.VMEM((8, ppad), jnp.float32)],
    )(ids, offs)
    pos2d = pos2d.reshape(F // 128, 128)

    fp_sorted = _make_permute_rows(F, H, gather=False)(fp, pos2d)

    out_sorted = pl.pallas_call(
        functools.partial(_seg_mm_kernel, tile_rows=TF),
        grid=(nt,),
        in_specs=[
            pl.BlockSpec(memory_space=pltpu.SMEM),
            pl.BlockSpec(memory_space=pltpu.SMEM),
            pl.BlockSpec((TF, H), lambda t: (t, 0)),
            pl.BlockSpec((num_ids, H, O), lambda t: (0, 0, 0)),
            pl.BlockSpec((num_ids, O), lambda t: (0, 0)),
        ],
        out_specs=pl.BlockSpec((TF, O), lambda t: (t, 0)),
        out_shape=jax.ShapeDtypeStruct((F, O), jnp.float32),
    )(offs, tlohi, fp_sorted, params.astype(jnp.bfloat16),
      bias.reshape(num_ids, O))

    out = _make_permute_rows(F, O, gather=True)(out_sorted, pos2d)
    return out
